# Initial kernel scaffold; baseline (speedup 1.0000x reference)
import jax
import jax.numpy as jnp
from jax.experimental import pallas as pl


def kernel(x, neighbor_idx, graph_ptr, W0, b0, W1, b1, Wc, bc):
    raise NotImplementedError("write your pallas kernel here")


# speedup vs baseline: 1.9934x; 1.9934x over previous
import jax
import jax.numpy as jnp
from jax import lax
from jax.experimental import pallas as pl
from jax.experimental.pallas import tpu as pltpu
from jax.experimental.pallas import tpu_sc as plsc

_L = 16


def _sc_gather_sum(xp, nbr2d, NP, D, DEG, NW, NPW):
    DL = D // _L
    GROUP = 128 // DEG
    SUPER = 32
    GPS = SUPER // GROUP
    NSUP = NPW // SUPER

    def body(x_hbm, nbr_hbm, out_hbm, idx_v, rows0, rows1, self_v, out_v,
             sem0, sem1):
        cid = lax.axis_index("c")
        sid = lax.axis_index("s")
        wid = sid * 2 + cid
        base = wid * NPW

        def super_body(s, carry):
            node0 = pl.multiple_of(base + s * SUPER, SUPER)
            row0 = pl.multiple_of(node0 // GROUP, GPS)
            pltpu.sync_copy(nbr_hbm.at[pl.ds(row0, GPS)], idx_v)
            pltpu.sync_copy(x_hbm.at[pl.ds(node0, SUPER)], self_v)
            rows = [rows0, rows1]
            sems = [sem0, sem1]
            cps = [None] * GPS
            cps[0] = pltpu.async_copy(x_hbm.at[idx_v.at[0]], rows0, sem0)
            for g in range(GPS):
                if g + 1 < GPS:
                    b = (g + 1) % 2
                    cps[g + 1] = pltpu.async_copy(
                        x_hbm.at[idx_v.at[g + 1]], rows[b], sems[b])
                cps[g].wait()
                rb = rows[g % 2]
                for i in range(GROUP):
                    nl = g * GROUP + i

                    def jbody(j, accs, _i=i, _rb=rb):
                        r = _i * DEG + j * 4
                        out = accs
                        for jj in range(4):
                            out = [a + _rb[r + jj, pl.ds(k * _L, _L)]
                                   for k, a in enumerate(out)]
                        return out

                    accs = [self_v[nl, pl.ds(k * _L, _L)] for k in range(DL)]
                    accs = lax.fori_loop(0, DEG // 4, jbody, accs)
                    for k in range(DL):
                        out_v[nl, pl.ds(k * _L, _L)] = accs[k]
            pltpu.sync_copy(out_v, out_hbm.at[pl.ds(node0, SUPER)])
            return carry

        lax.fori_loop(0, NSUP, super_body, 0)

    mesh = plsc.VectorSubcoreMesh(core_axis_name="c", subcore_axis_name="s")
    return pl.kernel(
        body,
        out_type=jax.ShapeDtypeStruct((NP, D), jnp.float32),
        mesh=mesh,
        scratch_types=[
            pltpu.VMEM((GPS, 128), jnp.int32),
            pltpu.VMEM((128, D), jnp.float32),
            pltpu.VMEM((128, D), jnp.float32),
            pltpu.VMEM((SUPER, D), jnp.float32),
            pltpu.VMEM((SUPER, D), jnp.float32),
            pltpu.SemaphoreType.DMA,
            pltpu.SemaphoreType.DMA,
        ],
    )(xp, nbr2d)


def _tc_dense(xp, agg, oh, W0, b0, Wc, bc, NP, D, G, C):
    BLK = 512
    nblk = NP // BLK
    hi = jax.lax.Precision.HIGHEST

    def body(x_ref, a_ref, oh_ref, w0_ref, b0_ref, wc_ref, bc_ref, out_ref,
             ge0, ge1):
        i = pl.program_id(0)

        @pl.when(i == 0)
        def _():
            ge0[...] = jnp.zeros_like(ge0)
            ge1[...] = jnp.zeros_like(ge1)

        xb = x_ref[...]
        ab = a_ref[...]
        ohb = oh_ref[...]
        h1 = jnp.maximum(
            lax.dot(ab, w0_ref[...], precision=hi,
                    preferred_element_type=jnp.float32) + b0_ref[...], 0.0)
        dn = (((0,), (0,)), ((), ()))
        ge0[...] += lax.dot_general(ohb, xb, dn, precision=hi,
                                    preferred_element_type=jnp.float32)
        ge1[...] += lax.dot_general(ohb, h1, dn, precision=hi,
                                    preferred_element_type=jnp.float32)

        @pl.when(i == nblk - 1)
        def _():
            p0 = lax.dot(ge0[...], wc_ref[...], precision=hi,
                         preferred_element_type=jnp.float32) + bc_ref[...]
            p1 = lax.dot(ge1[...], wc_ref[...], precision=hi,
                         preferred_element_type=jnp.float32) + bc_ref[...]
            out_ref[...] = jnp.stack([p0, p1])

    return pl.pallas_call(
        body,
        grid=(nblk,),
        in_specs=[
            pl.BlockSpec((BLK, D), lambda i: (i, 0)),
            pl.BlockSpec((BLK, D), lambda i: (i, 0)),
            pl.BlockSpec((BLK, G), lambda i: (i, 0)),
            pl.BlockSpec((D, D), lambda i: (0, 0)),
            pl.BlockSpec((1, D), lambda i: (0, 0)),
            pl.BlockSpec((D, C), lambda i: (0, 0)),
            pl.BlockSpec((1, C), lambda i: (0, 0)),
        ],
        out_specs=pl.BlockSpec((2, G, C), lambda i: (0, 0, 0)),
        out_shape=jax.ShapeDtypeStruct((2, G, C), jnp.float32),
        scratch_shapes=[
            pltpu.VMEM((G, D), jnp.float32),
            pltpu.VMEM((G, D), jnp.float32),
        ],
    )(xp, agg, oh, W0, b0.reshape(1, D), Wc, bc.reshape(1, C))


def kernel(x, neighbor_idx, graph_ptr, W0, b0, W1, b1, Wc, bc):
    N, D = x.shape
    DEG = neighbor_idx.shape[1]
    G = graph_ptr.shape[0] - 1
    C = Wc.shape[1]
    NW = 32
    SUPER = 32
    NPW = -(-N // NW)
    NPW = -(-NPW // SUPER) * SUPER
    NP = NW * NPW

    xp = jnp.zeros((NP, D), x.dtype).at[:N].set(x)
    nbrp = jnp.zeros((NP, DEG), jnp.int32).at[:N].set(neighbor_idx)
    nbr2d = nbrp.reshape(NP * DEG // 128, 128)

    agg = _sc_gather_sum(xp, nbr2d, NP, D, DEG, NW, NPW)

    seg = jnp.searchsorted(graph_ptr[1:], jnp.arange(N, dtype=jnp.int32),
                           side="right")
    oh = (seg[:, None] == jnp.arange(G)[None, :]).astype(jnp.float32)
    ohp = jnp.zeros((NP, G), jnp.float32).at[:N].set(oh)

    return _tc_dense(xp, agg, ohp, W0, b0, Wc, bc, NP, D, G, C)


# speedup vs baseline: 2.1809x; 1.0941x over previous
import jax
import jax.numpy as jnp
from jax import lax
from jax.experimental import pallas as pl
from jax.experimental.pallas import tpu as pltpu
from jax.experimental.pallas import tpu_sc as plsc

_L = 16


def _sc_gather_sum(xp, nbr2d, NP, D, DEG, NW, NPW):
    DL = D // _L
    GROUP = 128 // DEG
    SUPER = 32
    GPS = SUPER // GROUP
    NSUP = NPW // SUPER

    NBUF = 4
    GPI = NBUF
    NPI = GPI * GROUP
    NG = NPW // GROUP
    NIT = NG // GPI

    def body(x_hbm, nbr_hbm, out_hbm, idx_v, rows0, rows1, rows2, rows3,
             out_v, sem0, sem1, sem2, sem3):
        cid = lax.axis_index("c")
        sid = lax.axis_index("s")
        wid = sid * 2 + cid
        base = wid * NPW
        rows = [rows0, rows1, rows2, rows3]
        sems = [sem0, sem1, sem2, sem3]

        pltpu.sync_copy(nbr_hbm.at[pl.ds(pl.multiple_of(base // GROUP, 8), NG)],
                        idx_v)
        for p in range(NBUF - 1):
            pltpu.async_copy(x_hbm.at[idx_v.at[p]], rows[p], sems[p])

        def iter_body(t, carry):
            g0 = t * GPI
            for b in range(GPI):
                g = g0 + b
                nxt = g + (NBUF - 1)

                @pl.when(nxt < NG)
                def _(nxt=nxt, b=b):
                    pltpu.async_copy(x_hbm.at[idx_v.at[nxt]],
                                     rows[(b + NBUF - 1) % NBUF],
                                     sems[(b + NBUF - 1) % NBUF])

                pltpu.make_async_copy(x_hbm.at[pl.ds(0, GROUP * DEG)],
                                      rows[b], sems[b]).wait()
                rb = rows[b]
                for i in range(GROUP):
                    nl = b * GROUP + i

                    def jbody(j, accs, _i=i, _rb=rb):
                        r = _i * DEG + j * 4
                        out = accs
                        for jj in range(4):
                            out = [a + _rb[r + jj, pl.ds(k * _L, _L)]
                                   for k, a in enumerate(out)]
                        return out

                    accs = [jnp.zeros((_L,), jnp.float32) for _ in range(DL)]
                    accs = lax.fori_loop(0, DEG // 4, jbody, accs)
                    for k in range(DL):
                        out_v[nl, pl.ds(k * _L, _L)] = accs[k]
            node0 = pl.multiple_of(base + t * NPI, NPI)
            pltpu.sync_copy(out_v, out_hbm.at[pl.ds(node0, NPI)])
            return carry

        lax.fori_loop(0, NIT, iter_body, 0)

    mesh = plsc.VectorSubcoreMesh(core_axis_name="c", subcore_axis_name="s")
    return pl.kernel(
        body,
        out_type=jax.ShapeDtypeStruct((NP, D), jnp.float32),
        mesh=mesh,
        scratch_types=[
            pltpu.VMEM((NG, 128), jnp.int32),
            pltpu.VMEM((GROUP * DEG, D), jnp.float32),
            pltpu.VMEM((GROUP * DEG, D), jnp.float32),
            pltpu.VMEM((GROUP * DEG, D), jnp.float32),
            pltpu.VMEM((GROUP * DEG, D), jnp.float32),
            pltpu.VMEM((NPI, D), jnp.float32),
            pltpu.SemaphoreType.DMA,
            pltpu.SemaphoreType.DMA,
            pltpu.SemaphoreType.DMA,
            pltpu.SemaphoreType.DMA,
        ],
    )(xp, nbr2d)


def _tc_dense(xp, agg, oh, W0, b0, Wc, bc, NP, D, G, C):
    BLK = 512
    nblk = NP // BLK
    hi = jax.lax.Precision.HIGHEST

    def body(x_ref, a_ref, oh_ref, w0_ref, b0_ref, wc_ref, bc_ref, out_ref,
             ge0, ge1):
        i = pl.program_id(0)

        @pl.when(i == 0)
        def _():
            ge0[...] = jnp.zeros_like(ge0)
            ge1[...] = jnp.zeros_like(ge1)

        xb = x_ref[...]
        ab = a_ref[...] + xb
        ohb = oh_ref[...]
        h1 = jnp.maximum(
            lax.dot(ab, w0_ref[...], precision=hi,
                    preferred_element_type=jnp.float32) + b0_ref[...], 0.0)
        dn = (((0,), (0,)), ((), ()))
        ge0[...] += lax.dot_general(ohb, xb, dn, precision=hi,
                                    preferred_element_type=jnp.float32)
        ge1[...] += lax.dot_general(ohb, h1, dn, precision=hi,
                                    preferred_element_type=jnp.float32)

        @pl.when(i == nblk - 1)
        def _():
            p0 = lax.dot(ge0[...], wc_ref[...], precision=hi,
                         preferred_element_type=jnp.float32) + bc_ref[...]
            p1 = lax.dot(ge1[...], wc_ref[...], precision=hi,
                         preferred_element_type=jnp.float32) + bc_ref[...]
            out_ref[...] = jnp.stack([p0, p1])

    return pl.pallas_call(
        body,
        grid=(nblk,),
        in_specs=[
            pl.BlockSpec((BLK, D), lambda i: (i, 0)),
            pl.BlockSpec((BLK, D), lambda i: (i, 0)),
            pl.BlockSpec((BLK, G), lambda i: (i, 0)),
            pl.BlockSpec((D, D), lambda i: (0, 0)),
            pl.BlockSpec((1, D), lambda i: (0, 0)),
            pl.BlockSpec((D, C), lambda i: (0, 0)),
            pl.BlockSpec((1, C), lambda i: (0, 0)),
        ],
        out_specs=pl.BlockSpec((2, G, C), lambda i: (0, 0, 0)),
        out_shape=jax.ShapeDtypeStruct((2, G, C), jnp.float32),
        scratch_shapes=[
            pltpu.VMEM((G, D), jnp.float32),
            pltpu.VMEM((G, D), jnp.float32),
        ],
    )(xp, agg, oh, W0, b0.reshape(1, D), Wc, bc.reshape(1, C))


def kernel(x, neighbor_idx, graph_ptr, W0, b0, W1, b1, Wc, bc):
    N, D = x.shape
    DEG = neighbor_idx.shape[1]
    G = graph_ptr.shape[0] - 1
    C = Wc.shape[1]
    NW = 32
    SUPER = 32
    NPW = -(-N // NW)
    NPW = -(-NPW // SUPER) * SUPER
    NP = NW * NPW

    xp = jnp.zeros((NP, D), x.dtype).at[:N].set(x)
    nbrp = jnp.zeros((NP, DEG), jnp.int32).at[:N].set(neighbor_idx)
    nbr2d = nbrp.reshape(NP * DEG // 128, 128)

    agg = _sc_gather_sum(xp, nbr2d, NP, D, DEG, NW, NPW)

    seg = jnp.searchsorted(graph_ptr[1:], jnp.arange(N, dtype=jnp.int32),
                           side="right")
    oh = (seg[:, None] == jnp.arange(G)[None, :]).astype(jnp.float32)
    ohp = jnp.zeros((NP, G), jnp.float32).at[:N].set(oh)

    return _tc_dense(xp, agg, ohp, W0, b0, Wc, bc, NP, D, G, C)


# speedup vs baseline: 8.4768x; 3.8868x over previous
import jax
import jax.numpy as jnp
from jax import lax
from jax.experimental import pallas as pl
from jax.experimental.pallas import tpu as pltpu
from jax.experimental.pallas import tpu_sc as plsc

_L = 16


def _sc_gather_sum(x, nbr2d, N, D, DEG, NW, NPW):
    DL = D // _L
    GROUP = 128 // DEG
    NBUF = 2
    GPI = NBUF
    NPI = GPI * GROUP
    NG = NPW // GROUP
    NIT = NG // GPI

    last_base = N - NPW
    assert last_base % NPI == 0

    A = 8 * (N // (16 * 8))
    TAIL = N - 16 * A

    def body(x_hbm, nbr_hbm, out_hbm, xs, idx_v, *rest):
        rows = list(rest[:NBUF])
        out_v = rest[NBUF]
        sems = list(rest[NBUF + 1:])
        cid = lax.axis_index("c")
        sid = lax.axis_index("s")
        wid = sid * 2 + cid
        last = wid == NW - 1
        base = pl.multiple_of(jnp.where(last, last_base, wid * NPW), NPI)

        st0 = pl.multiple_of(sid * A, 8)
        pltpu.sync_copy(x_hbm.at[pl.ds(st0, A)], xs.at[pl.ds(st0, A)])
        if TAIL:
            @pl.when(sid == 15)
            def _():
                pltpu.sync_copy(x_hbm.at[pl.ds(16 * A, TAIL)],
                                xs.at[pl.ds(16 * A, TAIL)])
        pltpu.sync_copy(nbr_hbm.at[pl.ds(pl.multiple_of(base * DEG, 8),
                                         NPW * DEG)], idx_v)
        plsc.subcore_barrier()
        for p in range(NBUF - 1):
            pltpu.async_copy(xs.at[idx_v.at[pl.ds(p * 128, 128)]],
                             rows[p], sems[p])

        def iter_body(t, carry):
            g0 = t * GPI
            for b in range(GPI):
                g = g0 + b
                nxt = g + (NBUF - 1)

                @pl.when(nxt < NG)
                def _(nxt=nxt, b=b):
                    pltpu.async_copy(
                        xs.at[idx_v.at[pl.ds(pl.multiple_of(nxt * 128, 128),
                                             128)]],
                        rows[(b + NBUF - 1) % NBUF],
                        sems[(b + NBUF - 1) % NBUF])

                pltpu.make_async_copy(xs.at[pl.ds(0, GROUP * DEG)],
                                      rows[b], sems[b]).wait()
                rb = rows[b]
                for i in range(GROUP):
                    nl = b * GROUP + i

                    def jbody(j, accs, _i=i, _rb=rb):
                        r = _i * DEG + j * 4
                        out = accs
                        for jj in range(4):
                            out = [a + _rb[r + jj, pl.ds(k * _L, _L)]
                                   for k, a in enumerate(out)]
                        return out

                    accs = [jnp.zeros((_L,), jnp.float32) for _ in range(DL)]
                    accs = lax.fori_loop(0, DEG // 4, jbody, accs)
                    for k in range(DL):
                        out_v[nl, pl.ds(k * _L, _L)] = accs[k]
            node0 = pl.multiple_of(base + t * NPI, NPI)
            pltpu.sync_copy(out_v, out_hbm.at[pl.ds(node0, NPI)])
            return carry

        lax.fori_loop(0, NIT, iter_body, 0)

    mesh = plsc.VectorSubcoreMesh(core_axis_name="c", subcore_axis_name="s")
    return pl.kernel(
        body,
        out_type=jax.ShapeDtypeStruct((N, D), jnp.float32),
        mesh=mesh,
        scratch_types=(
            [pltpu.VMEM_SHARED((N, D), jnp.float32),
             pltpu.VMEM((NPW * DEG,), jnp.int32)]
            + [pltpu.VMEM((GROUP * DEG, D), jnp.float32)
               for _ in range(NBUF)]
            + [pltpu.VMEM((NPI, D), jnp.float32)]
            + [pltpu.SemaphoreType.DMA for _ in range(NBUF)]
        ),
    )(x, nbr2d)


def _tc_dense(x, agg, ptr_lo, ptr_hi, W0, b0, Wc, bc, N, D, G, C):
    BLK = 400
    nblk = N // BLK
    hi = jax.lax.Precision.HIGHEST

    def body(x_ref, a_ref, lo_ref, hi_ref, w0_ref, b0_ref, wc_ref, bc_ref,
             out_ref, ge0, ge1):
        i = pl.program_id(0)

        @pl.when(i == 0)
        def _():
            ge0[...] = jnp.zeros_like(ge0)
            ge1[...] = jnp.zeros_like(ge1)

        xb = x_ref[...]
        ab = a_ref[...] + xb
        rid = lax.broadcasted_iota(jnp.int32, (BLK, G), 0) + i * BLK
        ohb = ((rid >= lo_ref[...]) & (rid < hi_ref[...])).astype(jnp.float32)
        h1 = jnp.maximum(
            lax.dot(ab, w0_ref[...], precision=hi,
                    preferred_element_type=jnp.float32) + b0_ref[...], 0.0)
        dn = (((0,), (0,)), ((), ()))
        ge0[...] += lax.dot_general(ohb, xb, dn, precision=hi,
                                    preferred_element_type=jnp.float32)
        ge1[...] += lax.dot_general(ohb, h1, dn, precision=hi,
                                    preferred_element_type=jnp.float32)

        @pl.when(i == nblk - 1)
        def _():
            p0 = lax.dot(ge0[...], wc_ref[...], precision=hi,
                         preferred_element_type=jnp.float32) + bc_ref[...]
            p1 = lax.dot(ge1[...], wc_ref[...], precision=hi,
                         preferred_element_type=jnp.float32) + bc_ref[...]
            out_ref[...] = jnp.stack([p0, p1])

    return pl.pallas_call(
        body,
        grid=(nblk,),
        in_specs=[
            pl.BlockSpec((BLK, D), lambda i: (i, 0)),
            pl.BlockSpec((BLK, D), lambda i: (i, 0)),
            pl.BlockSpec((1, G), lambda i: (0, 0)),
            pl.BlockSpec((1, G), lambda i: (0, 0)),
            pl.BlockSpec((D, D), lambda i: (0, 0)),
            pl.BlockSpec((1, D), lambda i: (0, 0)),
            pl.BlockSpec((D, C), lambda i: (0, 0)),
            pl.BlockSpec((1, C), lambda i: (0, 0)),
        ],
        out_specs=pl.BlockSpec((2, G, C), lambda i: (0, 0, 0)),
        out_shape=jax.ShapeDtypeStruct((2, G, C), jnp.float32),
        scratch_shapes=[
            pltpu.VMEM((G, D), jnp.float32),
            pltpu.VMEM((G, D), jnp.float32),
        ],
    )(x, agg, ptr_lo, ptr_hi, W0, b0.reshape(1, D), Wc, bc.reshape(1, C))


def kernel(x, neighbor_idx, graph_ptr, W0, b0, W1, b1, Wc, bc):
    N, D = x.shape
    DEG = neighbor_idx.shape[1]
    G = graph_ptr.shape[0] - 1
    C = Wc.shape[1]
    NW = 32
    NPW = -(-N // NW)
    NPW = -(-NPW // 32) * 32

    nbr1d = neighbor_idx.reshape(N * DEG)
    agg = _sc_gather_sum(x, nbr1d, N, D, DEG, NW, NPW)

    ptr = graph_ptr.astype(jnp.int32)
    ptr_lo = ptr[:-1].reshape(1, G)
    ptr_hi = ptr[1:].reshape(1, G)
    return _tc_dense(x, agg, ptr_lo, ptr_hi, W0, b0, Wc, bc, N, D, G, C)
